# Initial kernel scaffold; baseline (speedup 1.0000x reference)
import jax
import jax.numpy as jnp
from jax.experimental import pallas as pl


def kernel(h_id_tensor, r_id_tensor, t_id_tensor, q_emb, entity_embs, num_non_text_entities, relation_embs, topic_entity_one_hot, non_text_emb, W1, b1, W2, b2):
    raise NotImplementedError("write your pallas kernel here")


# speedup vs baseline: 12.3946x; 12.3946x over previous
import functools

import jax
import jax.numpy as jnp
from jax import lax
from jax.experimental import pallas as pl
from jax.experimental.pallas import tpu as pltpu
from jax.experimental.pallas import tpu_sc as plsc

_N = 10000
_NTEXT = 8000
_E = 640000
_D = 128
_R = 512
_NROW = 79
_NP = _NROW * 128
_NW = 32
_EPT = _E // _NW
_ECH = 128
_NCH = 160
_EPTE = _ECH * _NCH
_EPAD = _EPTE * _NW
_SUP = 8

_mesh = plsc.VectorSubcoreMesh(core_axis_name="c", subcore_axis_name="s",
                               num_cores=2, num_subcores=16)
_f32 = jnp.float32


def _wid():
    return lax.axis_index("s") * 2 + lax.axis_index("c")


def _dde_body(with_counts, *args):
    if with_counts:
        (h_hbm, t_hbm, xf_hbm, z_hbm,
         accT_hbm, accH_hbm, cntT_hbm, cntH_hbm,
         xf0, xf1, aT0, aT1, aH0, aH1, cT, cH, h_v, t_v) = args
        xr0, xr1 = xf0, xf1
    else:
        (h_hbm, t_hbm, xf_hbm, xr_hbm, z_hbm, accT_hbm, accH_hbm,
         xf0, xf1, xr0, xr1, aT0, aT1, aH0, aH1, h_v, t_v) = args
        cT = cH = None
    w = _wid()
    base = w * _EPT
    pltpu.sync_copy(xf_hbm.at[0], xf0)
    pltpu.sync_copy(xf_hbm.at[1], xf1)
    if not with_counts:
        pltpu.sync_copy(xr_hbm.at[0], xr0)
        pltpu.sync_copy(xr_hbm.at[1], xr1)
    accs = [aT0, aT1, aH0, aH1] + ([cT, cH] if with_counts else [])
    for r in accs:
        pltpu.sync_copy(z_hbm, r)
    pltpu.sync_copy(h_hbm.at[pl.ds(base, _EPT)], h_v)
    pltpu.sync_copy(t_hbm.at[pl.ds(base, _EPT)], t_v)
    ones = jnp.ones((16,), _f32)

    def body(i, carry):
        hv = h_v[pl.ds(i * 16, 16)]
        tv = t_v[pl.ds(i * 16, 16)]
        plsc.addupdate_scatter(aT0, [tv], plsc.load_gather(xf0, [hv]))
        plsc.addupdate_scatter(aT1, [tv], plsc.load_gather(xf1, [hv]))
        plsc.addupdate_scatter(aH0, [hv], plsc.load_gather(xr0, [tv]))
        plsc.addupdate_scatter(aH1, [hv], plsc.load_gather(xr1, [tv]))
        if with_counts:
            plsc.addupdate_scatter(cT, [tv], ones)
            plsc.addupdate_scatter(cH, [hv], ones)
        return carry

    lax.fori_loop(0, _EPT // 16, body, 0)
    pltpu.sync_copy(aT0, accT_hbm.at[w, 0])
    pltpu.sync_copy(aT1, accT_hbm.at[w, 1])
    pltpu.sync_copy(aH0, accH_hbm.at[w, 0])
    pltpu.sync_copy(aH1, accH_hbm.at[w, 1])
    if with_counts:
        pltpu.sync_copy(cT, cntT_hbm.at[w])
        pltpu.sync_copy(cH, cntH_hbm.at[w])


def _make_dde(with_counts):
    tab = pltpu.VMEM((_NP,), _f32)
    ev = pltpu.VMEM((_EPT,), jnp.int32)
    n_buf = 8
    out = [jax.ShapeDtypeStruct((_NW, 2, _NP), _f32)] * 2
    if with_counts:
        out += [jax.ShapeDtypeStruct((_NW, _NP), _f32)] * 2
    return pl.kernel(
        functools.partial(_dde_body, with_counts),
        out_type=tuple(out),
        mesh=_mesh,
        compiler_params=pltpu.CompilerParams(needs_layout_passes=False),
        scratch_types=[tab] * n_buf + [ev, ev],
    )


def _combine1_body(aT_ref, aH_ref, cT_ref, cH_ref, d1_ref, d3_ref, cTo, cHo):
    cT = jnp.maximum(jnp.sum(cT_ref[...], axis=0), 1.0)
    cH = jnp.maximum(jnp.sum(cH_ref[...], axis=0), 1.0)
    d1_ref[...] = jnp.sum(aT_ref[...], axis=0) / jnp.concatenate([cT, cT])
    d3_ref[...] = jnp.sum(aH_ref[...], axis=0) / jnp.concatenate([cH, cH])
    cTo[...] = cT
    cHo[...] = cH


def _combine2_body(aT_ref, aH_ref, cT_ref, cH_ref, d2_ref, d4_ref):
    cT = cT_ref[...]
    cH = cH_ref[...]
    d2_ref[...] = jnp.sum(aT_ref[...], axis=0) / jnp.concatenate([cT, cT])
    d4_ref[...] = jnp.sum(aH_ref[...], axis=0) / jnp.concatenate([cH, cH])


def _dense_body(ent_ref, nte_ref, coff_ref, q_ref, rel_ref, W1_ref, b1_ref,
                hpe_ref, A_ref, B_ref, C_ref):
    W1 = W1_ref[...]
    h_e = jnp.concatenate(
        [ent_ref[...], jnp.broadcast_to(nte_ref[...], (_NP - _NTEXT, _D))],
        axis=0) + coff_ref[...]
    hpe = hpe_ref[...]
    A_ref[...] = (jnp.dot(h_e, W1[_D:2 * _D], preferred_element_type=_f32)
                  + jnp.dot(hpe, W1[2 * _D:2 * _D + 10],
                            preferred_element_type=_f32))
    B_ref[...] = (jnp.dot(h_e, W1[2 * _D + 138:3 * _D + 138],
                          preferred_element_type=_f32)
                  + jnp.dot(hpe, W1[3 * _D + 138:3 * _D + 148],
                            preferred_element_type=_f32))
    C_ref[...] = (jnp.dot(rel_ref[...], W1[_D + 138:2 * _D + 138],
                          preferred_element_type=_f32)
                  + jnp.dot(q_ref[...], W1[:_D], preferred_element_type=_f32)
                  + b1_ref[...][None, :])


def _edge_body(h_hbm, r_hbm, t_hbm, A_hbm, B_hbm, C_hbm, w2_hbm, b2_hbm,
               out_hbm, hI, rI, tI, ab0, ab1, cb0, cb1, bb0, bb1,
               w2v, b2v, outv, sA0, sC0, sB0, sA1, sC1, sB1):
    w = _wid()
    ebase = w * _EPTE
    pltpu.sync_copy(w2_hbm, w2v)
    b2v[...] = jnp.zeros((16,), _f32)
    pltpu.sync_copy(b2_hbm, b2v.at[pl.ds(0, 1)])
    w2 = [w2v[pl.ds(16 * j, 16)] for j in range(8)]
    bvec = b2v[...]
    lane = lax.iota(jnp.int32, 16)
    lane0 = lane == 0
    bufs = ((ab0, cb0, bb0, sA0, sC0, sB0), (ab1, cb1, bb1, sA1, sC1, sB1))

    def stage_idx(sc):
        off = ebase + sc * (_SUP * _ECH)
        pltpu.sync_copy(h_hbm.at[pl.ds(off, _SUP * _ECH)], hI)
        pltpu.sync_copy(r_hbm.at[pl.ds(off, _SUP * _ECH)], rI)
        pltpu.sync_copy(t_hbm.at[pl.ds(off, _SUP * _ECH)], tI)

    def issue(g, b):
        a, cc, bb, sa, sc_, sb = bufs[b]
        off = (g % _SUP) * _ECH
        pltpu.async_copy(A_hbm.at[hI.at[pl.ds(off, _ECH)]], a, sa)
        pltpu.async_copy(C_hbm.at[rI.at[pl.ds(off, _ECH)]], cc, sc_)
        pltpu.async_copy(B_hbm.at[tI.at[pl.ds(off, _ECH)]], bb, sb)

    def wait(b):
        a, cc, bb, sa, sc_, sb = bufs[b]
        pltpu.make_async_copy(A_hbm.at[hI.at[pl.ds(0, _ECH)]], a, sa).wait()
        pltpu.make_async_copy(C_hbm.at[rI.at[pl.ds(0, _ECH)]], cc, sc_).wait()
        pltpu.make_async_copy(B_hbm.at[tI.at[pl.ds(0, _ECH)]], bb, sb).wait()

    def compute(g, b):
        a, cc, bb, *_ = bufs[b]

        def edge(e, carry):
            acc = bvec
            for j in range(8):
                sl = pl.ds(16 * j, 16)
                v = a[e, sl] + cc[e, sl] + bb[e, sl]
                acc = acc + jnp.maximum(v, 0.0) * w2[j]
            s = jnp.sum(acc)
            plsc.store_scatter(outv, [jnp.full((16,), g * _ECH + e, jnp.int32)],
                               jnp.full((16,), s, _f32), mask=lane0)
            return carry

        lax.fori_loop(0, _ECH, edge, 0)

    stage_idx(0)
    issue(0, 0)
    def step(i, carry):
        for b in (0, 1):
            g = 2 * i + b
            wait(b)

            @pl.when(g + 1 < _NCH)
            def _():
                @pl.when((g + 1) % _SUP == 0)
                def _():
                    stage_idx((g + 1) // _SUP)
                issue(g + 1, 1 - b)

            compute(g, b)
        return carry

    lax.fori_loop(0, _NCH // 2, step, 0)
    pltpu.sync_copy(outv, out_hbm.at[pl.ds(ebase, _EPTE)])


_edge_kernel = pl.kernel(
    _edge_body,
    out_type=jax.ShapeDtypeStruct((_EPAD,), _f32),
    mesh=_mesh,
    compiler_params=pltpu.CompilerParams(needs_layout_passes=False),
    scratch_types=(
        [pltpu.VMEM((_SUP * _ECH,), jnp.int32)] * 3
        + [pltpu.VMEM((_ECH, _D), _f32)] * 6
        + [pltpu.VMEM((_D,), _f32), pltpu.VMEM((16,), _f32),
           pltpu.VMEM((_EPTE,), _f32)]
        + [pltpu.SemaphoreType.DMA] * 6
    ),
)


def kernel(h_id_tensor, r_id_tensor, t_id_tensor, q_emb, entity_embs,
           num_non_text_entities, relation_embs, topic_entity_one_hot,
           non_text_emb, W1, b1, W2, b2):
    h_id = h_id_tensor.astype(jnp.int32)
    r_id = r_id_tensor.astype(jnp.int32)
    t_id = t_id_tensor.astype(jnp.int32)
    topic = topic_entity_one_hot.astype(_f32)

    zeros = jnp.zeros((_NP,), _f32)
    pad_n = jnp.zeros((2, _NP - _N), _f32)
    topic_pl = jnp.concatenate([topic.T, pad_n], axis=1)

    aT, aH, cT, cH = _make_dde(True)(h_id, t_id, topic_pl, zeros)
    d1, d3, cTc, cHc = pl.pallas_call(
        _combine1_body,
        out_shape=[jax.ShapeDtypeStruct((2 * _NP,), _f32)] * 2
        + [jax.ShapeDtypeStruct((_NP,), _f32)] * 2,
    )(aT.reshape(_NW, 2 * _NP), aH.reshape(_NW, 2 * _NP),
      cT.reshape(_NW, _NP), cH.reshape(_NW, _NP))

    aT2, aH2 = _make_dde(False)(h_id, t_id, d1.reshape(2, _NP),
                                d3.reshape(2, _NP), zeros)
    d2, d4 = pl.pallas_call(
        _combine2_body,
        out_shape=[jax.ShapeDtypeStruct((2 * _NP,), _f32)] * 2,
    )(aT2.reshape(_NW, 2 * _NP), aH2.reshape(_NW, 2 * _NP), cTc, cHc)

    tpad = jnp.concatenate([topic, jnp.zeros((_NP - _N, 2), _f32)], axis=0)
    hpe = jnp.concatenate(
        [tpad] + [x.reshape(2, _NP).T for x in (d1, d2, d3, d4)], axis=1)

    coff = (jnp.asarray(num_non_text_entities, _f32)
            - (_N - _NTEXT)).reshape(1, 1)
    A, B, C = pl.pallas_call(
        _dense_body,
        out_shape=[jax.ShapeDtypeStruct((_NP, _D), _f32)] * 2
        + [jax.ShapeDtypeStruct((_R, _D), _f32)],
    )(entity_embs, non_text_emb, coff, q_emb, relation_embs, W1, b1, hpe)

    pad_e = jnp.zeros((_EPAD - _E,), jnp.int32)
    pred = _edge_kernel(
        jnp.concatenate([h_id, pad_e]), jnp.concatenate([r_id, pad_e]),
        jnp.concatenate([t_id, pad_e]), A, B, C,
        W2.reshape(_D), b2.reshape(1))
    return pred[:_E].reshape(_E, 1)


# speedup vs baseline: 19.8816x; 1.6041x over previous
import functools

import jax
import jax.numpy as jnp
from jax import lax
from jax.experimental import pallas as pl
from jax.experimental.pallas import tpu as pltpu
from jax.experimental.pallas import tpu_sc as plsc

_N = 10000
_NTEXT = 8000
_E = 640000
_D = 128
_R = 512
_NROW = 79
_NP = _NROW * 128
_NW = 32
_EPT = _E // _NW
_ECH = 32
_NCH = 640
_EPTE = _ECH * _NCH
_EPAD = _EPTE * _NW
_SUP = 8

_mesh = plsc.VectorSubcoreMesh(core_axis_name="c", subcore_axis_name="s",
                               num_cores=2, num_subcores=16)
_f32 = jnp.float32


def _wid():
    return lax.axis_index("s") * 2 + lax.axis_index("c")


def _dde_body(with_counts, *args):
    if with_counts:
        (h_hbm, t_hbm, xf_hbm, z_hbm,
         accT_hbm, accH_hbm, cntT_hbm, cntH_hbm,
         xf0, xf1, aT0, aT1, aH0, aH1, cT, cH, h_v, t_v) = args
        xr0, xr1 = xf0, xf1
    else:
        (h_hbm, t_hbm, xf_hbm, xr_hbm, z_hbm, accT_hbm, accH_hbm,
         xf0, xf1, xr0, xr1, aT0, aT1, aH0, aH1, h_v, t_v) = args
        cT = cH = None
    w = _wid()
    base = w * _EPT
    pltpu.sync_copy(xf_hbm.at[0], xf0)
    pltpu.sync_copy(xf_hbm.at[1], xf1)
    if not with_counts:
        pltpu.sync_copy(xr_hbm.at[0], xr0)
        pltpu.sync_copy(xr_hbm.at[1], xr1)
    accs = [aT0, aT1, aH0, aH1] + ([cT, cH] if with_counts else [])
    for r in accs:
        pltpu.sync_copy(z_hbm, r)
    pltpu.sync_copy(h_hbm.at[pl.ds(base, _EPT)], h_v)
    pltpu.sync_copy(t_hbm.at[pl.ds(base, _EPT)], t_v)
    ones = jnp.ones((16,), _f32)

    def body(i, carry):
        hv = h_v[pl.ds(i * 16, 16)]
        tv = t_v[pl.ds(i * 16, 16)]
        plsc.addupdate_scatter(aT0, [tv], plsc.load_gather(xf0, [hv]))
        plsc.addupdate_scatter(aT1, [tv], plsc.load_gather(xf1, [hv]))
        plsc.addupdate_scatter(aH0, [hv], plsc.load_gather(xr0, [tv]))
        plsc.addupdate_scatter(aH1, [hv], plsc.load_gather(xr1, [tv]))
        if with_counts:
            plsc.addupdate_scatter(cT, [tv], ones)
            plsc.addupdate_scatter(cH, [hv], ones)
        return carry

    lax.fori_loop(0, _EPT // 16, body, 0)
    pltpu.sync_copy(aT0, accT_hbm.at[w, 0])
    pltpu.sync_copy(aT1, accT_hbm.at[w, 1])
    pltpu.sync_copy(aH0, accH_hbm.at[w, 0])
    pltpu.sync_copy(aH1, accH_hbm.at[w, 1])
    if with_counts:
        pltpu.sync_copy(cT, cntT_hbm.at[w])
        pltpu.sync_copy(cH, cntH_hbm.at[w])


def _make_dde(with_counts):
    tab = pltpu.VMEM((_NP,), _f32)
    ev = pltpu.VMEM((_EPT,), jnp.int32)
    n_buf = 8
    out = [jax.ShapeDtypeStruct((_NW, 2, _NP), _f32)] * 2
    if with_counts:
        out += [jax.ShapeDtypeStruct((_NW, _NP), _f32)] * 2
    return pl.kernel(
        functools.partial(_dde_body, with_counts),
        out_type=tuple(out),
        mesh=_mesh,
        compiler_params=pltpu.CompilerParams(needs_layout_passes=False),
        scratch_types=[tab] * n_buf + [ev, ev],
    )


def _combine1_body(aT_ref, aH_ref, cT_ref, cH_ref, d1_ref, d3_ref, cTo, cHo):
    cT = jnp.maximum(jnp.sum(cT_ref[...], axis=0), 1.0)
    cH = jnp.maximum(jnp.sum(cH_ref[...], axis=0), 1.0)
    d1_ref[...] = jnp.sum(aT_ref[...], axis=0) / jnp.concatenate([cT, cT])
    d3_ref[...] = jnp.sum(aH_ref[...], axis=0) / jnp.concatenate([cH, cH])
    cTo[...] = cT
    cHo[...] = cH


def _combine2_body(aT_ref, aH_ref, cT_ref, cH_ref, d2_ref, d4_ref):
    cT = cT_ref[...]
    cH = cH_ref[...]
    d2_ref[...] = jnp.sum(aT_ref[...], axis=0) / jnp.concatenate([cT, cT])
    d4_ref[...] = jnp.sum(aH_ref[...], axis=0) / jnp.concatenate([cH, cH])


def _dense_body(ent_ref, nte_ref, coff_ref, q_ref, rel_ref, W1_ref, b1_ref,
                hpe_ref, A_ref, B_ref, C_ref):
    W1 = W1_ref[...]
    h_e = jnp.concatenate(
        [ent_ref[...], jnp.broadcast_to(nte_ref[...], (_NP - _NTEXT, _D))],
        axis=0) + coff_ref[...]
    hpe = hpe_ref[...]
    A_ref[...] = (jnp.dot(h_e, W1[_D:2 * _D], preferred_element_type=_f32)
                  + jnp.dot(hpe, W1[2 * _D:2 * _D + 10],
                            preferred_element_type=_f32)).astype(jnp.bfloat16)
    B_ref[...] = (jnp.dot(h_e, W1[2 * _D + 138:3 * _D + 138],
                          preferred_element_type=_f32)
                  + jnp.dot(hpe, W1[3 * _D + 138:3 * _D + 148],
                            preferred_element_type=_f32)).astype(jnp.bfloat16)
    C_ref[...] = (jnp.dot(rel_ref[...], W1[_D + 138:2 * _D + 138],
                          preferred_element_type=_f32)
                  + jnp.dot(q_ref[...], W1[:_D], preferred_element_type=_f32)
                  + b1_ref[...][None, :]).astype(jnp.bfloat16)


def _edge_body(h_hbm, r_hbm, t_hbm, T_hbm, C2_hbm, w2_hbm, b2_hbm,
               out_hbm, T_sh, C_sh, hI, rI, tI, ab0, ab1, cb0, cb1,
               bb0, bb1, w2v, b2v, ov0, ov1, sA0, sC0, sB0, sA1, sC1, sB1,
               sO0, sO1):
    w = _wid()
    ebase = w * _EPTE

    @pl.when(lax.axis_index("s") == 0)
    def _():
        pltpu.sync_copy(T_hbm, T_sh)
        pltpu.sync_copy(C2_hbm, C_sh)

    plsc.subcore_barrier()
    pltpu.sync_copy(w2_hbm, w2v)
    b2v[...] = jnp.zeros((16,), _f32)
    pltpu.sync_copy(b2_hbm, b2v.at[pl.ds(0, 1)])
    w2 = [w2v[pl.ds(16 * j, 16)] for j in range(8)]
    bvec = b2v[...]
    lane = lax.iota(jnp.int32, 16)
    lane0 = lane == 0
    bufs = ((ab0, cb0, bb0, sA0, sC0, sB0), (ab1, cb1, bb1, sA1, sC1, sB1))
    obufs = ((ov0, sO0), (ov1, sO1))

    def stage_idx(sc):
        off = ebase + sc * (_SUP * _ECH)
        pltpu.sync_copy(h_hbm.at[pl.ds(off, _SUP * _ECH)], hI)
        pltpu.sync_copy(r_hbm.at[pl.ds(off, _SUP * _ECH)], rI)
        pltpu.sync_copy(t_hbm.at[pl.ds(off, _SUP * _ECH)], tI)

    def issue(g, b):
        a, cc, bb, sa, sc_, sb = bufs[b]
        off = (g % _SUP) * _ECH
        pltpu.async_copy(T_sh.at[hI.at[pl.ds(off, _ECH)]], a, sa)
        pltpu.async_copy(C_sh.at[rI.at[pl.ds(off, _ECH)]], cc, sc_)
        pltpu.async_copy(T_sh.at[tI.at[pl.ds(off, _ECH)]], bb, sb)

    def wait(b):
        a, cc, bb, sa, sc_, sb = bufs[b]
        pltpu.make_async_copy(T_sh.at[hI.at[pl.ds(0, _ECH)]], a, sa).wait()
        pltpu.make_async_copy(C_sh.at[rI.at[pl.ds(0, _ECH)]], cc, sc_).wait()
        pltpu.make_async_copy(T_sh.at[tI.at[pl.ds(0, _ECH)]], bb, sb).wait()

    himask = jnp.full((16,), -65536, jnp.int32)

    def _halves(wv):
        return (plsc.bitcast(wv << 16, _f32),
                plsc.bitcast(wv & himask, _f32))

    def compute(g, b):
        a, cc, bb, *_ = bufs[b]
        ov, so = obufs[b]

        @pl.when(g >= 2)
        def _():
            pltpu.make_async_copy(
                ov, out_hbm.at[pl.ds(ebase + (g - 2) * _ECH, _ECH)], so).wait()

        def edge(e, carry):
            acc = bvec
            for j in range(4):
                sl = pl.ds(16 * j, 16)
                alo, ahi = _halves(a[e, sl])
                clo, chi = _halves(cc[e, sl])
                blo, bhi = _halves(bb[e, pl.ds(64 + 16 * j, 16)])
                acc = acc + jnp.maximum(alo + clo + blo, 0.0) * w2[2 * j]
                acc = acc + jnp.maximum(ahi + chi + bhi, 0.0) * w2[2 * j + 1]
            s = jnp.sum(acc)
            plsc.store_scatter(ov, [jnp.full((16,), e, jnp.int32)],
                               jnp.full((16,), s, _f32), mask=lane0)
            return carry

        lax.fori_loop(0, _ECH, edge, 0)
        pltpu.async_copy(ov, out_hbm.at[pl.ds(ebase + g * _ECH, _ECH)], so)

    stage_idx(0)
    issue(0, 0)
    def step(i, carry):
        for b in (0, 1):
            g = 2 * i + b
            wait(b)

            @pl.when(g + 1 < _NCH)
            def _():
                @pl.when((g + 1) % _SUP == 0)
                def _():
                    stage_idx((g + 1) // _SUP)
                issue(g + 1, 1 - b)

            compute(g, b)
        return carry

    lax.fori_loop(0, _NCH // 2, step, 0)
    for q in (0, 1):
        ov, so = obufs[q]
        pltpu.make_async_copy(
            ov, out_hbm.at[pl.ds(ebase + (_NCH - 2 + q) * _ECH, _ECH)],
            so).wait()


_edge_kernel = pl.kernel(
    _edge_body,
    out_type=jax.ShapeDtypeStruct((_EPAD,), _f32),
    mesh=_mesh,
    compiler_params=pltpu.CompilerParams(needs_layout_passes=False),
    scratch_types=(
        [pltpu.VMEM_SHARED((_NP, _D), jnp.int32)]
        + [pltpu.VMEM_SHARED((_R, _D), jnp.int32)]
        + [pltpu.VMEM((_SUP * _ECH,), jnp.int32)] * 3
        + [pltpu.VMEM((_ECH, _D), jnp.int32)] * 6
        + [pltpu.VMEM((_D,), _f32), pltpu.VMEM((16,), _f32)]
        + [pltpu.VMEM((_ECH,), _f32)] * 2
        + [pltpu.SemaphoreType.DMA] * 8
    ),
)


def kernel(h_id_tensor, r_id_tensor, t_id_tensor, q_emb, entity_embs,
           num_non_text_entities, relation_embs, topic_entity_one_hot,
           non_text_emb, W1, b1, W2, b2):
    h_id = h_id_tensor.astype(jnp.int32)
    r_id = r_id_tensor.astype(jnp.int32)
    t_id = t_id_tensor.astype(jnp.int32)
    topic = topic_entity_one_hot.astype(_f32)

    zeros = jnp.zeros((_NP,), _f32)
    pad_n = jnp.zeros((2, _NP - _N), _f32)
    topic_pl = jnp.concatenate([topic.T, pad_n], axis=1)

    aT, aH, cT, cH = _make_dde(True)(h_id, t_id, topic_pl, zeros)
    d1, d3, cTc, cHc = pl.pallas_call(
        _combine1_body,
        out_shape=[jax.ShapeDtypeStruct((2 * _NP,), _f32)] * 2
        + [jax.ShapeDtypeStruct((_NP,), _f32)] * 2,
    )(aT.reshape(_NW, 2 * _NP), aH.reshape(_NW, 2 * _NP),
      cT.reshape(_NW, _NP), cH.reshape(_NW, _NP))

    aT2, aH2 = _make_dde(False)(h_id, t_id, d1.reshape(2, _NP),
                                d3.reshape(2, _NP), zeros)
    d2, d4 = pl.pallas_call(
        _combine2_body,
        out_shape=[jax.ShapeDtypeStruct((2 * _NP,), _f32)] * 2,
    )(aT2.reshape(_NW, 2 * _NP), aH2.reshape(_NW, 2 * _NP), cTc, cHc)

    tpad = jnp.concatenate([topic, jnp.zeros((_NP - _N, 2), _f32)], axis=0)
    hpe = jnp.concatenate(
        [tpad] + [x.reshape(2, _NP).T for x in (d1, d2, d3, d4)], axis=1)

    coff = (jnp.asarray(num_non_text_entities, _f32)
            - (_N - _NTEXT)).reshape(1, 1)
    A, B, C = pl.pallas_call(
        _dense_body,
        out_shape=[jax.ShapeDtypeStruct((_NP, _D), jnp.bfloat16)] * 2
        + [jax.ShapeDtypeStruct((_R, _D), jnp.bfloat16)],
    )(entity_embs, non_text_emb, coff, q_emb, relation_embs, W1, b1, hpe)

    A, B, C = (lax.bitcast_convert_type(
        x.reshape(x.shape[0], _D // 2, 2), jnp.int32) for x in (A, B, C))
    T = jnp.concatenate([A, B], axis=1)
    C2 = jnp.concatenate([C, C], axis=1)
    pad_e = jnp.zeros((_EPAD - _E,), jnp.int32)
    w2f = W2.reshape(_D)
    w2_perm = jnp.concatenate(
        [jnp.concatenate([w2f[32 * j:32 * j + 32:2],
                          w2f[32 * j + 1:32 * j + 32:2]]) for j in range(4)])
    pred = _edge_kernel(
        jnp.concatenate([h_id, pad_e]), jnp.concatenate([r_id, pad_e]),
        jnp.concatenate([t_id, pad_e]), T, C2,
        w2_perm, b2.reshape(1))
    return pred[:_E].reshape(_E, 1)
